# baseline (speedup 1.0000x reference)
import functools

import jax
import jax.numpy as jnp
from jax import lax
from jax.experimental import pallas as pl
from jax.experimental.pallas import tpu as pltpu
from jax.experimental.pallas import tpu_sc as plsc

N = 50000
E = 800000
H = 80
HH = 40
L = 10
NC = 2
NS = 16
NP = 50176
ROWS_T = NP // NS
EPAD = 802816
BLK = 2000
NBLK = N // BLK

CH = 64
KCH = EPAD // NS // CH
CHD = 128
KCH_D = EPAD // (NC * NS) // CHD


def _deg_body(dstf, ones, z1, d0, d1, dst_v, ones_v, deg_sh):
    c = lax.axis_index("c")
    s = lax.axis_index("s")
    sl = pl.ds(s * ROWS_T, ROWS_T)
    pltpu.sync_copy(z1, deg_sh.at[sl])
    pltpu.sync_copy(ones, ones_v)
    dbase = (c * NS + s) * (KCH_D * CHD)
    plsc.subcore_barrier()

    def body(j, carry):
        pltpu.sync_copy(dstf.at[pl.ds(dbase + j * CHD, CHD)], dst_v.at[0])
        pltpu.sync_copy(ones_v, deg_sh.at[dst_v.at[0]], add=True)
        return carry

    lax.fori_loop(0, KCH_D, body, 0)
    plsc.subcore_barrier()

    @pl.when(c == 0)
    def _():
        pltpu.sync_copy(deg_sh.at[sl], d0.at[sl])

    @pl.when(c == 1)
    def _():
        pltpu.sync_copy(deg_sh.at[sl], d1.at[sl])


def _agg_body(hn_flat, comb, z2, a0, a1, idx_v, rows_v, agg_sh, sem):
    c = lax.axis_index("c")
    s = lax.axis_index("s")
    sl = pl.ds(s * ROWS_T, ROWS_T)
    pltpu.sync_copy(z2, agg_sh.at[sl])
    rbase = (c * NS + s) * KCH * 2
    plsc.subcore_barrier()

    i0, i1 = idx_v.at[0], idx_v.at[1]
    r0, r1 = rows_v.at[0], rows_v.at[1]
    pltpu.sync_copy(comb.at[pl.ds(rbase, 2)], i0)
    pltpu.async_copy(hn_flat.at[i0.at[0]], r0, sem)

    def body(jj, carry):
        j0 = jj * 2
        pltpu.make_async_copy(hn_flat.at[i0.at[0]], r0, sem).wait()
        pltpu.sync_copy(comb.at[pl.ds(rbase + (j0 + 1) * 2, 2)], i1)
        pltpu.async_copy(hn_flat.at[i1.at[0]], r1, sem)
        pltpu.sync_copy(r0, agg_sh.at[i0.at[1]], add=True)
        pltpu.make_async_copy(hn_flat.at[i1.at[0]], r1, sem).wait()

        @pl.when(jj < KCH // 2 - 1)
        def _():
            pltpu.sync_copy(comb.at[pl.ds(rbase + (j0 + 2) * 2, 2)], i0)
            pltpu.async_copy(hn_flat.at[i0.at[0]], r0, sem)

        pltpu.sync_copy(r1, agg_sh.at[i1.at[1]], add=True)
        return carry

    lax.fori_loop(0, KCH // 2, body, 0)
    plsc.subcore_barrier()

    @pl.when(c == 0)
    def _():
        pltpu.sync_copy(agg_sh.at[sl], a0.at[sl])

    @pl.when(c == 1)
    def _():
        pltpu.sync_copy(agg_sh.at[sl], a1.at[sl])


@functools.cache
def _sc_kernels():
    mesh = plsc.VectorSubcoreMesh(core_axis_name="c", subcore_axis_name="s",
                                  num_cores=NC, num_subcores=NS)
    params = pltpu.CompilerParams(use_tc_tiling_on_sc=False)
    deg = pl.kernel(
        _deg_body,
        out_type=(jax.ShapeDtypeStruct((NP,), jnp.float32),
                  jax.ShapeDtypeStruct((NP,), jnp.float32)),
        mesh=mesh,
        compiler_params=params,
        scratch_types=[
            pltpu.VMEM((1, CHD), jnp.int32),
            pltpu.VMEM((CHD,), jnp.float32),
            pltpu.VMEM_SHARED((NP,), jnp.float32),
        ],
    )
    agg = pl.kernel(
        _agg_body,
        out_type=(jax.ShapeDtypeStruct((NP, HH), jnp.float32),
                  jax.ShapeDtypeStruct((NP, HH), jnp.float32)),
        mesh=mesh,
        compiler_params=params,
        scratch_types=[
            pltpu.VMEM((2, 2, CH), jnp.int32),
            pltpu.VMEM((2, CH, HH), jnp.float32),
            pltpu.VMEM_SHARED((NP, HH), jnp.float32),
            pltpu.SemaphoreType.DMA,
        ],
    )
    return deg, agg


def _prep_body(x_ref, ew_ref, eb_ref, d0_ref, d1_ref,
               h_ref, hn2_ref, norm_ref):
    h = x_ref[...] @ ew_ref[...] + eb_ref[...]
    deg = jnp.maximum(d0_ref[...] + d1_ref[...], 1.0)
    nrm = lax.rsqrt(deg)
    h_ref[...] = h
    hn = h * nrm
    hn2_ref[...] = jnp.stack([hn[:, :HH], hn[:, HH:]], axis=0)
    norm_ref[...] = nrm


_prep = pl.pallas_call(
    _prep_body,
    grid=(NBLK,),
    in_specs=[
        pl.BlockSpec((BLK, 6), lambda i: (i, 0)),
        pl.BlockSpec((6, H), lambda i: (0, 0)),
        pl.BlockSpec((1, H), lambda i: (0, 0)),
        pl.BlockSpec((BLK, 1), lambda i: (i, 0)),
        pl.BlockSpec((BLK, 1), lambda i: (i, 0)),
    ],
    out_specs=[
        pl.BlockSpec((BLK, H), lambda i: (i, 0)),
        pl.BlockSpec((2, BLK, HH), lambda i: (0, i, 0)),
        pl.BlockSpec((BLK, 1), lambda i: (i, 0)),
    ],
    out_shape=[
        jax.ShapeDtypeStruct((N, H), jnp.float32),
        jax.ShapeDtypeStruct((2, N, HH), jnp.float32),
        jax.ShapeDtypeStruct((N, 1), jnp.float32),
    ],
)


def _mm_body(a0_ref, a1_ref, norm_ref, w_ref, b_ref,
             t_ref, stats_ref, acc):
    i = pl.program_id(0)
    agg = jnp.concatenate([a0_ref[...], a1_ref[...]], axis=1)
    t = (agg * norm_ref[...]) @ w_ref[...] + b_ref[...]
    t_ref[...] = t

    @pl.when(i == 0)
    def _():
        acc[...] = jnp.zeros_like(acc)

    acc[0:1, :] += jnp.sum(t, axis=0, keepdims=True)
    acc[1:2, :] += jnp.sum(t * t, axis=0, keepdims=True)

    @pl.when(i == NBLK - 1)
    def _():
        stats_ref[...] = acc[...]


_mm = pl.pallas_call(
    _mm_body,
    grid=(NBLK,),
    in_specs=[
        pl.BlockSpec((BLK, HH), lambda i: (i, 0)),
        pl.BlockSpec((BLK, HH), lambda i: (i, 0)),
        pl.BlockSpec((BLK, 1), lambda i: (i, 0)),
        pl.BlockSpec((H, H), lambda i: (0, 0)),
        pl.BlockSpec((1, H), lambda i: (0, 0)),
    ],
    out_specs=[
        pl.BlockSpec((BLK, H), lambda i: (i, 0)),
        pl.BlockSpec((2, H), lambda i: (0, 0)),
    ],
    out_shape=[
        jax.ShapeDtypeStruct((N, H), jnp.float32),
        jax.ShapeDtypeStruct((2, H), jnp.float32),
    ],
    scratch_shapes=[pltpu.VMEM((2, H), jnp.float32)],
)


def _bn_core(t_ref, stats_ref, g_ref, be_ref, hin_ref):
    st = stats_ref[...]
    mean = st[0:1, :] * (1.0 / N)
    var = st[1:2, :] * (1.0 / N) - mean * mean
    inv = lax.rsqrt(var + 1e-5)
    y = (t_ref[...] - mean) * (inv * g_ref[...]) + be_ref[...]
    y = jnp.maximum(y, 0.0)
    return hin_ref[...] + y


def _bn_body(t_ref, stats_ref, g_ref, be_ref, hin_ref, norm_ref,
             h_ref, hn2_ref):
    h = _bn_core(t_ref, stats_ref, g_ref, be_ref, hin_ref)
    h_ref[...] = h
    hn = h * norm_ref[...]
    hn2_ref[...] = jnp.stack([hn[:, :HH], hn[:, HH:]], axis=0)


_bn = pl.pallas_call(
    _bn_body,
    grid=(NBLK,),
    in_specs=[
        pl.BlockSpec((BLK, H), lambda i: (i, 0)),
        pl.BlockSpec((2, H), lambda i: (0, 0)),
        pl.BlockSpec((1, H), lambda i: (0, 0)),
        pl.BlockSpec((1, H), lambda i: (0, 0)),
        pl.BlockSpec((BLK, H), lambda i: (i, 0)),
        pl.BlockSpec((BLK, 1), lambda i: (i, 0)),
    ],
    out_specs=[
        pl.BlockSpec((BLK, H), lambda i: (i, 0)),
        pl.BlockSpec((2, BLK, HH), lambda i: (0, i, 0)),
    ],
    out_shape=[
        jax.ShapeDtypeStruct((N, H), jnp.float32),
        jax.ShapeDtypeStruct((2, N, HH), jnp.float32),
    ],
)


def _bn_last_body(t_ref, stats_ref, g_ref, be_ref, hin_ref, hg_ref, mx):
    i = pl.program_id(0)
    h = _bn_core(t_ref, stats_ref, g_ref, be_ref, hin_ref)

    @pl.when(i == 0)
    def _():
        mx[...] = jnp.full_like(mx, -jnp.inf)

    mx[...] = jnp.maximum(mx[...], jnp.max(h, axis=0, keepdims=True))

    @pl.when(i == NBLK - 1)
    def _():
        hg_ref[...] = mx[...]


_bn_last = pl.pallas_call(
    _bn_last_body,
    grid=(NBLK,),
    in_specs=[
        pl.BlockSpec((BLK, H), lambda i: (i, 0)),
        pl.BlockSpec((2, H), lambda i: (0, 0)),
        pl.BlockSpec((1, H), lambda i: (0, 0)),
        pl.BlockSpec((1, H), lambda i: (0, 0)),
        pl.BlockSpec((BLK, H), lambda i: (i, 0)),
    ],
    out_specs=pl.BlockSpec((1, H), lambda i: (0, 0)),
    out_shape=jax.ShapeDtypeStruct((1, H), jnp.float32),
    scratch_shapes=[pltpu.VMEM((1, H), jnp.float32)],
)


def _mlp_body(hg_ref, w1, b1, w2, b2, w3, b3, out_ref):
    y = jnp.maximum(hg_ref[...] @ w1[...] + b1[...], 0.0)
    y = jnp.maximum(y @ w2[...] + b2[...], 0.0)
    out_ref[...] = y @ w3[...] + b3[...]


_mlp = pl.pallas_call(
    _mlp_body,
    out_shape=jax.ShapeDtypeStruct((1, 3), jnp.float32),
)


def kernel(x, edge_index, emb_W, emb_b, gcn_W, gcn_b, bn_gamma, bn_beta,
           mlp_W1, mlp_b1, mlp_W2, mlp_b2, mlp_W3, mlp_b3):
    src = edge_index[0]
    dst = edge_index[1]
    npad = EPAD - E
    pad_src = (jnp.arange(npad, dtype=jnp.int32) % 64)
    pad_dst = N + (jnp.arange(npad, dtype=jnp.int32) % (NP - N))
    src_p = jnp.concatenate([src, pad_src])
    dst_p = jnp.concatenate([dst, pad_dst])
    src2 = jnp.concatenate([src_p, src_p + N]).reshape(NC * NS, KCH, CH)
    dstb = jnp.broadcast_to(dst_p.reshape(1, NS, KCH, CH),
                            (NC, NS, KCH, CH)).reshape(NC * NS, KCH, CH)
    comb = jnp.stack([src2, dstb], axis=2).reshape(NC * NS * KCH * 2, CH)
    ones_c = jnp.ones((CHD,), jnp.float32)
    z1 = jnp.zeros((ROWS_T,), jnp.float32)
    z2 = jnp.zeros((ROWS_T, HH), jnp.float32)

    _deg_kernel, _agg_kernel = _sc_kernels()
    d0, d1 = _deg_kernel(dst_p, ones_c, z1)
    h, hn2, norm = _prep(x, emb_W, emb_b.reshape(1, H),
                         d0[:N].reshape(N, 1), d1[:N].reshape(N, 1))
    for i in range(L):
        a0, a1 = _agg_kernel(hn2.reshape(2 * N, HH), comb, z2)
        t, stats = _mm(a0, a1, norm, gcn_W[i], gcn_b[i].reshape(1, H))
        gi = bn_gamma[i].reshape(1, H)
        bi = bn_beta[i].reshape(1, H)
        if i < L - 1:
            h, hn2 = _bn(t, stats, gi, bi, h, norm)
        else:
            hg = _bn_last(t, stats, gi, bi, h)
    return _mlp(hg, mlp_W1, mlp_b1.reshape(1, HH),
                mlp_W2, mlp_b2.reshape(1, H // 4),
                mlp_W3, mlp_b3.reshape(1, 3))


# speedup vs baseline: 1.7200x; 1.7200x over previous
import functools

import jax
import jax.numpy as jnp
from jax import lax
from jax.experimental import pallas as pl
from jax.experimental.pallas import tpu as pltpu
from jax.experimental.pallas import tpu_sc as plsc

N = 50000
E = 800000
H = 80
HH = 40
L = 10
NC = 2
NS = 16
NP = 50048
ROWS_T = NP // NS
EPAD = 802816
BLK = 2000
NBLK = N // BLK

CH = 128
KCH = EPAD // NS // CH
CHD = 128
KCH_D = EPAD // (NC * NS) // CHD


def _deg_body(dstf, ones, z1, d0, d1, dst_v, ones_v, deg_sh):
    c = lax.axis_index("c")
    s = lax.axis_index("s")
    sl = pl.ds(s * ROWS_T, ROWS_T)
    pltpu.sync_copy(z1, deg_sh.at[sl])
    pltpu.sync_copy(ones, ones_v)
    dbase = (c * NS + s) * (KCH_D * CHD)
    plsc.subcore_barrier()

    def body(j, carry):
        pltpu.sync_copy(dstf.at[pl.ds(dbase + j * CHD, CHD)], dst_v.at[0])
        pltpu.sync_copy(ones_v, deg_sh.at[dst_v.at[0]], add=True)
        return carry

    lax.fori_loop(0, KCH_D, body, 0)
    plsc.subcore_barrier()

    @pl.when(c == 0)
    def _():
        pltpu.sync_copy(deg_sh.at[sl], d0.at[sl])

    @pl.when(c == 1)
    def _():
        pltpu.sync_copy(deg_sh.at[sl], d1.at[sl])


def _agg_body(hn_flat, comb, z2, a0, a1, idx_v, rows_v, agg_sh, sem):
    c = lax.axis_index("c")
    s = lax.axis_index("s")
    sl = pl.ds(s * ROWS_T, ROWS_T)
    pltpu.sync_copy(z2, agg_sh.at[sl])
    rbase = (c * NS + s) * KCH * 2
    plsc.subcore_barrier()

    i0, i1 = idx_v.at[0], idx_v.at[1]
    pltpu.sync_copy(comb.at[pl.ds(rbase, 2)], i0)

    def body(jj, carry):
        j0 = jj * 2
        pltpu.async_copy(hn_flat.at[i0.at[0]], rows_v, sem)
        pltpu.sync_copy(comb.at[pl.ds(rbase + (j0 + 1) * 2, 2)], i1)
        pltpu.make_async_copy(hn_flat.at[i0.at[0]], rows_v, sem).wait()
        pltpu.sync_copy(rows_v, agg_sh.at[i0.at[1]], add=True)
        pltpu.async_copy(hn_flat.at[i1.at[0]], rows_v, sem)

        @pl.when(jj < KCH // 2 - 1)
        def _():
            pltpu.sync_copy(comb.at[pl.ds(rbase + (j0 + 2) * 2, 2)], i0)

        pltpu.make_async_copy(hn_flat.at[i1.at[0]], rows_v, sem).wait()
        pltpu.sync_copy(rows_v, agg_sh.at[i1.at[1]], add=True)
        return carry

    lax.fori_loop(0, KCH // 2, body, 0)
    plsc.subcore_barrier()

    @pl.when(c == 0)
    def _():
        pltpu.sync_copy(agg_sh.at[sl], a0.at[sl])

    @pl.when(c == 1)
    def _():
        pltpu.sync_copy(agg_sh.at[sl], a1.at[sl])


@functools.cache
def _sc_kernels():
    mesh = plsc.VectorSubcoreMesh(core_axis_name="c", subcore_axis_name="s",
                                  num_cores=NC, num_subcores=NS)
    params = pltpu.CompilerParams(use_tc_tiling_on_sc=False)
    deg = pl.kernel(
        _deg_body,
        out_type=(jax.ShapeDtypeStruct((NP,), jnp.float32),
                  jax.ShapeDtypeStruct((NP,), jnp.float32)),
        mesh=mesh,
        compiler_params=params,
        scratch_types=[
            pltpu.VMEM((1, CHD), jnp.int32),
            pltpu.VMEM((CHD,), jnp.float32),
            pltpu.VMEM_SHARED((NP,), jnp.float32),
        ],
    )
    agg = pl.kernel(
        _agg_body,
        out_type=(jax.ShapeDtypeStruct((NP, HH), jnp.float32),
                  jax.ShapeDtypeStruct((NP, HH), jnp.float32)),
        mesh=mesh,
        compiler_params=params,
        scratch_types=[
            pltpu.VMEM((2, 2, CH), jnp.int32),
            pltpu.VMEM((CH, HH), jnp.float32),
            pltpu.VMEM_SHARED((NP, HH), jnp.float32),
            pltpu.SemaphoreType.DMA,
        ],
    )
    return deg, agg


def _prep_body(x_ref, ew_ref, eb_ref, d0_ref, d1_ref,
               h_ref, hn2_ref, norm_ref):
    h = x_ref[...] @ ew_ref[...] + eb_ref[...]
    deg = jnp.maximum(d0_ref[...] + d1_ref[...], 1.0)
    nrm = lax.rsqrt(deg)
    h_ref[...] = h
    hn = h * nrm
    hn2_ref[...] = jnp.stack([hn[:, :HH], hn[:, HH:]], axis=0)
    norm_ref[...] = nrm


_prep = pl.pallas_call(
    _prep_body,
    grid=(NBLK,),
    in_specs=[
        pl.BlockSpec((BLK, 6), lambda i: (i, 0)),
        pl.BlockSpec((6, H), lambda i: (0, 0)),
        pl.BlockSpec((1, H), lambda i: (0, 0)),
        pl.BlockSpec((BLK, 1), lambda i: (i, 0)),
        pl.BlockSpec((BLK, 1), lambda i: (i, 0)),
    ],
    out_specs=[
        pl.BlockSpec((BLK, H), lambda i: (i, 0)),
        pl.BlockSpec((2, BLK, HH), lambda i: (0, i, 0)),
        pl.BlockSpec((BLK, 1), lambda i: (i, 0)),
    ],
    out_shape=[
        jax.ShapeDtypeStruct((N, H), jnp.float32),
        jax.ShapeDtypeStruct((2, N, HH), jnp.float32),
        jax.ShapeDtypeStruct((N, 1), jnp.float32),
    ],
)


def _mm_body(a0_ref, a1_ref, norm_ref, w_ref, b_ref,
             t_ref, stats_ref, acc):
    i = pl.program_id(0)
    agg = jnp.concatenate([a0_ref[...], a1_ref[...]], axis=1)
    t = (agg * norm_ref[...]) @ w_ref[...] + b_ref[...]
    t_ref[...] = t

    @pl.when(i == 0)
    def _():
        acc[...] = jnp.zeros_like(acc)

    acc[0:1, :] += jnp.sum(t, axis=0, keepdims=True)
    acc[1:2, :] += jnp.sum(t * t, axis=0, keepdims=True)

    @pl.when(i == NBLK - 1)
    def _():
        stats_ref[...] = acc[...]


_mm = pl.pallas_call(
    _mm_body,
    grid=(NBLK,),
    in_specs=[
        pl.BlockSpec((BLK, HH), lambda i: (i, 0)),
        pl.BlockSpec((BLK, HH), lambda i: (i, 0)),
        pl.BlockSpec((BLK, 1), lambda i: (i, 0)),
        pl.BlockSpec((H, H), lambda i: (0, 0)),
        pl.BlockSpec((1, H), lambda i: (0, 0)),
    ],
    out_specs=[
        pl.BlockSpec((BLK, H), lambda i: (i, 0)),
        pl.BlockSpec((2, H), lambda i: (0, 0)),
    ],
    out_shape=[
        jax.ShapeDtypeStruct((N, H), jnp.float32),
        jax.ShapeDtypeStruct((2, H), jnp.float32),
    ],
    scratch_shapes=[pltpu.VMEM((2, H), jnp.float32)],
)


def _bn_core(t_ref, stats_ref, g_ref, be_ref, hin_ref):
    st = stats_ref[...]
    mean = st[0:1, :] * (1.0 / N)
    var = st[1:2, :] * (1.0 / N) - mean * mean
    inv = lax.rsqrt(var + 1e-5)
    y = (t_ref[...] - mean) * (inv * g_ref[...]) + be_ref[...]
    y = jnp.maximum(y, 0.0)
    return hin_ref[...] + y


def _bn_body(t_ref, stats_ref, g_ref, be_ref, hin_ref, norm_ref,
             h_ref, hn2_ref):
    h = _bn_core(t_ref, stats_ref, g_ref, be_ref, hin_ref)
    h_ref[...] = h
    hn = h * norm_ref[...]
    hn2_ref[...] = jnp.stack([hn[:, :HH], hn[:, HH:]], axis=0)


_bn = pl.pallas_call(
    _bn_body,
    grid=(NBLK,),
    in_specs=[
        pl.BlockSpec((BLK, H), lambda i: (i, 0)),
        pl.BlockSpec((2, H), lambda i: (0, 0)),
        pl.BlockSpec((1, H), lambda i: (0, 0)),
        pl.BlockSpec((1, H), lambda i: (0, 0)),
        pl.BlockSpec((BLK, H), lambda i: (i, 0)),
        pl.BlockSpec((BLK, 1), lambda i: (i, 0)),
    ],
    out_specs=[
        pl.BlockSpec((BLK, H), lambda i: (i, 0)),
        pl.BlockSpec((2, BLK, HH), lambda i: (0, i, 0)),
    ],
    out_shape=[
        jax.ShapeDtypeStruct((N, H), jnp.float32),
        jax.ShapeDtypeStruct((2, N, HH), jnp.float32),
    ],
)


def _bn_last_body(t_ref, stats_ref, g_ref, be_ref, hin_ref, hg_ref, mx):
    i = pl.program_id(0)
    h = _bn_core(t_ref, stats_ref, g_ref, be_ref, hin_ref)

    @pl.when(i == 0)
    def _():
        mx[...] = jnp.full_like(mx, -jnp.inf)

    mx[...] = jnp.maximum(mx[...], jnp.max(h, axis=0, keepdims=True))

    @pl.when(i == NBLK - 1)
    def _():
        hg_ref[...] = mx[...]


_bn_last = pl.pallas_call(
    _bn_last_body,
    grid=(NBLK,),
    in_specs=[
        pl.BlockSpec((BLK, H), lambda i: (i, 0)),
        pl.BlockSpec((2, H), lambda i: (0, 0)),
        pl.BlockSpec((1, H), lambda i: (0, 0)),
        pl.BlockSpec((1, H), lambda i: (0, 0)),
        pl.BlockSpec((BLK, H), lambda i: (i, 0)),
    ],
    out_specs=pl.BlockSpec((1, H), lambda i: (0, 0)),
    out_shape=jax.ShapeDtypeStruct((1, H), jnp.float32),
    scratch_shapes=[pltpu.VMEM((1, H), jnp.float32)],
)


def _mlp_body(hg_ref, w1, b1, w2, b2, w3, b3, out_ref):
    y = jnp.maximum(hg_ref[...] @ w1[...] + b1[...], 0.0)
    y = jnp.maximum(y @ w2[...] + b2[...], 0.0)
    out_ref[...] = y @ w3[...] + b3[...]


_mlp = pl.pallas_call(
    _mlp_body,
    out_shape=jax.ShapeDtypeStruct((1, 3), jnp.float32),
)


def kernel(x, edge_index, emb_W, emb_b, gcn_W, gcn_b, bn_gamma, bn_beta,
           mlp_W1, mlp_b1, mlp_W2, mlp_b2, mlp_W3, mlp_b3):
    src = edge_index[0]
    dst = edge_index[1]
    npad = EPAD - E
    pad_src = (jnp.arange(npad, dtype=jnp.int32) % 64)
    pad_dst = N + (jnp.arange(npad, dtype=jnp.int32) % (NP - N))
    src_p = jnp.concatenate([src, pad_src])
    dst_p = jnp.concatenate([dst, pad_dst])
    src2 = jnp.concatenate([src_p, src_p + N]).reshape(NC * NS, KCH, CH)
    dstb = jnp.broadcast_to(dst_p.reshape(1, NS, KCH, CH),
                            (NC, NS, KCH, CH)).reshape(NC * NS, KCH, CH)
    comb = jnp.stack([src2, dstb], axis=2).reshape(NC * NS * KCH * 2, CH)
    ones_c = jnp.ones((CHD,), jnp.float32)
    z1 = jnp.zeros((ROWS_T,), jnp.float32)
    z2 = jnp.zeros((ROWS_T, HH), jnp.float32)

    _deg_kernel, _agg_kernel = _sc_kernels()
    d0, d1 = _deg_kernel(dst_p, ones_c, z1)
    h, hn2, norm = _prep(x, emb_W, emb_b.reshape(1, H),
                         d0[:N].reshape(N, 1), d1[:N].reshape(N, 1))
    for i in range(L):
        a0, a1 = _agg_kernel(hn2.reshape(2 * N, HH), comb, z2)
        t, stats = _mm(a0, a1, norm, gcn_W[i], gcn_b[i].reshape(1, H))
        gi = bn_gamma[i].reshape(1, H)
        bi = bn_beta[i].reshape(1, H)
        if i < L - 1:
            h, hn2 = _bn(t, stats, gi, bi, h, norm)
        else:
            hg = _bn_last(t, stats, gi, bi, h)
    return _mlp(hg, mlp_W1, mlp_b1.reshape(1, HH),
                mlp_W2, mlp_b2.reshape(1, H // 4),
                mlp_W3, mlp_b3.reshape(1, 3))
